# Initial kernel scaffold; baseline (speedup 1.0000x reference)
import jax
import jax.numpy as jnp
from jax.experimental import pallas as pl


def kernel(x):
    raise NotImplementedError("write your pallas kernel here")


# speedup vs baseline: 3.1065x; 3.1065x over previous
import jax
import jax.numpy as jnp
from jax import lax
from jax.experimental import pallas as pl
from jax.experimental.pallas import tpu as pltpu
from jax.experimental.pallas import tpu_sc as plsc

_TOPK = 64
_B = 128
_N = 8192
_L = 16
_NVEC = _N // _L
_NSEG = 4
_SEGV = _NVEC // _NSEG
_NC = 2
_NS = 16
_NW = _NC * _NS
_ROWS_PER_W = _B // _NW


def _vsort(v):
    return jnp.sort(v)


def _vrev(v):
    return lax.rev(v, (0,))


def _clean32(u, v):
    return _vsort(jnp.minimum(u, v)), _vsort(jnp.maximum(u, v))


def _merge16(a, b):
    rb = _vrev(b)
    return _vsort(jnp.minimum(a, rb)), _vsort(jnp.maximum(a, rb))


def _merge32(a0, a1, b0, b1):
    rb0, rb1 = _vrev(b1), _vrev(b0)
    l0, l1 = jnp.minimum(a0, rb0), jnp.minimum(a1, rb1)
    h0, h1 = jnp.maximum(a0, rb0), jnp.maximum(a1, rb1)
    p0, p1 = _clean32(l0, l1)
    q0, q1 = _clean32(h0, h1)
    return p0, p1, q0, q1


def _sort64(c0, c1, c2, c3):
    a0, a1 = _merge16(_vsort(c0), _vsort(c1))
    b0, b1 = _merge16(_vsort(c2), _vsort(c3))
    return _merge32(a0, a1, b0, b1)


def _merge_top64(s, c):
    t0 = jnp.maximum(s[0], _vrev(c[3]))
    t1 = jnp.maximum(s[1], _vrev(c[2]))
    t2 = jnp.maximum(s[2], _vrev(c[1]))
    t3 = jnp.maximum(s[3], _vrev(c[0]))
    l0, l1 = jnp.minimum(t0, t2), jnp.minimum(t1, t3)
    h0, h1 = jnp.maximum(t0, t2), jnp.maximum(t1, t3)
    p0, p1 = _clean32(l0, l1)
    q0, q1 = _clean32(h0, h1)
    return p0, p1, q0, q1


def _sc_body(x_hbm, out_hbm, row_v, cand_v, outrow_v):
    wid = lax.axis_index("s") * _NC + lax.axis_index("c")
    iota = lax.iota(jnp.int32, _L)
    ninf = jnp.full((_L,), -jnp.inf, jnp.float32)

    for r in range(_ROWS_PER_W):
        row = wid * _ROWS_PER_W + r
        pltpu.sync_copy(x_hbm.at[row], row_v)

        def p1_body(i, ms):
            m0, m1, m2, m3 = ms
            base = i * _L
            v0 = row_v[pl.ds(base, _L)]
            v1 = row_v[pl.ds(base + _SEGV * _L, _L)]
            v2 = row_v[pl.ds(base + 2 * _SEGV * _L, _L)]
            v3 = row_v[pl.ds(base + 3 * _SEGV * _L, _L)]
            return (jnp.maximum(m0, v0), jnp.maximum(m1, v1),
                    jnp.maximum(m2, v2), jnp.maximum(m3, v3))

        m0, m1, m2, m3 = lax.fori_loop(0, _SEGV, p1_body,
                                       (ninf, ninf, ninf, ninf))
        t = jnp.minimum(jnp.minimum(m0, m1), jnp.minimum(m2, m3))
        thr = jnp.full((_L,), jnp.min(t), jnp.float32)

        def p2_body(i, off):
            v = row_v[pl.ds(i * _L, _L)]
            msk = v >= thr
            cs = plsc.cumsum(msk.astype(jnp.int32))
            plsc.store_scatter(cand_v, [off + cs - 1], v, mask=msk)
            return off + plsc.all_reduce_population_count(msk)

        off = lax.fori_loop(0, _NVEC, p2_body, jnp.zeros((_L,), jnp.int32))

        for j in range(_TOPK // _L):
            plsc.store_scatter(cand_v, [off + (j * _L) + iota], ninf)
        c_s = jnp.max(off)
        nchunks = lax.shift_right_logical(c_s + (_TOPK - 1), 6)

        def p3_body(j, buf):
            base = jnp.full((_L,), j * _TOPK, jnp.int32) + iota
            c0 = plsc.load_gather(cand_v, [base])
            c1 = plsc.load_gather(cand_v, [base + _L])
            c2 = plsc.load_gather(cand_v, [base + 2 * _L])
            c3 = plsc.load_gather(cand_v, [base + 3 * _L])
            ch = _sort64(c0, c1, c2, c3)
            return _merge_top64(buf, ch)

        buf = lax.fori_loop(0, nchunks, p3_body, (ninf, ninf, ninf, ninf))

        outrow_v[pl.ds(0, _L)] = _vrev(buf[3])
        outrow_v[pl.ds(_L, _L)] = _vrev(buf[2])
        outrow_v[pl.ds(2 * _L, _L)] = _vrev(buf[1])
        outrow_v[pl.ds(3 * _L, _L)] = _vrev(buf[0])
        pltpu.sync_copy(outrow_v, out_hbm.at[row])


def kernel(x):
    run = pl.kernel(
        _sc_body,
        out_type=jax.ShapeDtypeStruct((_B, _TOPK), jnp.float32),
        mesh=plsc.VectorSubcoreMesh(core_axis_name="c", subcore_axis_name="s",
                                    num_cores=_NC, num_subcores=_NS),
        scratch_types=[
            pltpu.VMEM((_N,), jnp.float32),
            pltpu.VMEM((_N + _TOPK,), jnp.float32),
            pltpu.VMEM((_TOPK,), jnp.float32),
        ],
        compiler_params=pltpu.CompilerParams(needs_layout_passes=False),
    )
    return run(x)
